# baseline (speedup 1.0000x reference)
import functools

import jax
import jax.numpy as jnp
from jax import lax
from jax.experimental import pallas as pl
from jax.experimental.pallas import tpu as pltpu
from jax.experimental.pallas import tpu_sc as plsc

_CHUNK = 32
_NBUF = 3
_DEPTH = 2


def _make_sc_broadcast(b, s, h, dtype):
    info = plsc.get_sparse_core_info()
    nc, ns = info.num_cores, info.num_subcores
    nw = nc * ns
    rows_per_w = s // nw
    n_chunks = rows_per_w // _CHUNK
    mesh = plsc.VectorSubcoreMesh(core_axis_name="c", subcore_axis_name="s")

    scratch = [pltpu.VMEM((_CHUNK, h), dtype) for _ in range(_NBUF)]
    scratch += [pltpu.SemaphoreType.DMA for _ in range(2 * _NBUF)]

    @functools.partial(
        pl.kernel,
        mesh=mesh,
        out_type=jax.ShapeDtypeStruct((b, s, h), dtype),
        scratch_types=scratch,
    )
    def sc_broadcast(table_hbm, out_hbm, *bufs_sems):
        bufs = bufs_sems[:_NBUF]
        rsems = bufs_sems[_NBUF : 2 * _NBUF]
        wsems = bufs_sems[2 * _NBUF :]
        wid = lax.axis_index("s") * nc + lax.axis_index("c")
        base = wid * rows_per_w

        reads = [None] * _NBUF
        writes = [[] for _ in range(_NBUF)]
        for c in range(min(_DEPTH, n_chunks)):
            reads[c % _NBUF] = pltpu.async_copy(
                table_hbm.at[pl.ds(base + c * _CHUNK, _CHUNK)],
                bufs[c % _NBUF],
                rsems[c % _NBUF],
            )
        for c in range(n_chunks):
            k = c % _NBUF
            reads[k].wait()
            lo = base + c * _CHUNK
            writes[k] = [
                pltpu.async_copy(bufs[k], out_hbm.at[bi, pl.ds(lo, _CHUNK)], wsems[k])
                for bi in range(b)
            ]
            nxt = c + _DEPTH
            if nxt < n_chunks:
                kn = nxt % _NBUF
                for w in writes[kn]:
                    w.wait()
                writes[kn] = []
                reads[kn] = pltpu.async_copy(
                    table_hbm.at[pl.ds(base + nxt * _CHUNK, _CHUNK)], bufs[kn], rsems[kn]
                )
        for ws in writes:
            for w in ws:
                w.wait()

    return sc_broadcast


def kernel(x, pos_embedding):
    b = x.shape[0]
    s, h = pos_embedding.shape
    return _make_sc_broadcast(b, s, h, pos_embedding.dtype)(pos_embedding)
